# Initial kernel scaffold; baseline (speedup 1.0000x reference)
import jax
import jax.numpy as jnp
from jax.experimental import pallas as pl


def kernel(x, edge_index, edge_attr, W_e, b_e, g_e, be_e, W_n, b_n, g_n, be_n):
    raise NotImplementedError("write your pallas kernel here")


# speedup vs baseline: 2.9600x; 2.9600x over previous
import functools

import jax
import jax.numpy as jnp
from jax import lax
from jax.experimental import pallas as pl
from jax.experimental.pallas import tpu as pltpu
from jax.experimental.pallas import tpu_sc as plsc

NC = 2
NS = 16
LANES = 16
NW = NC * NS


def _proj_body(x_ref, w_ref, o_ref):
    o_ref[...] = jnp.dot(x_ref[...], w_ref[...],
                         preferred_element_type=jnp.float32)


def _make_gather_diff(N, E, D, CH):
    e_pt = E // NW
    nch = e_pt // CH
    mesh = plsc.VectorSubcoreMesh(core_axis_name="c", subcore_axis_name="s",
                                  num_cores=NC, num_subcores=NS)

    @functools.partial(
        pl.kernel,
        out_type=jax.ShapeDtypeStruct((E, D), jnp.float32),
        mesh=mesh,
        scratch_types=[
            pltpu.VMEM((CH,), jnp.int32),
            pltpu.VMEM((CH,), jnp.int32),
            pltpu.VMEM((CH, D), jnp.float32),
            pltpu.VMEM((CH, D), jnp.float32),
            pltpu.SemaphoreType.DMA,
            pltpu.SemaphoreType.DMA,
        ],
    )
    def gather_diff(u_hbm, row_hbm, col_hbm, out_hbm,
                    idx_r, idx_c, buf_r, buf_c, sem_r, sem_c):
        wid = lax.axis_index("s") * NC + lax.axis_index("c")
        base0 = wid * e_pt

        def chunk(i, _):
            base = base0 + i * CH
            pltpu.sync_copy(row_hbm.at[pl.ds(base, CH)], idx_r)
            pltpu.sync_copy(col_hbm.at[pl.ds(base, CH)], idx_c)
            cp_r = pltpu.async_copy(u_hbm.at[idx_r], buf_r, sem_r)
            cp_c = pltpu.async_copy(u_hbm.at[idx_c], buf_c, sem_c)
            cp_r.wait()
            cp_c.wait()

            def sub_row(j, _):
                for k in range(D // LANES):
                    sl = pl.ds(k * LANES, LANES)
                    buf_r[j, sl] = buf_r[j, sl] - buf_c[j, sl]
                return ()

            lax.fori_loop(0, CH, sub_row, (), unroll=2)
            pltpu.sync_copy(buf_r, out_hbm.at[pl.ds(base, CH)])
            return ()

        lax.fori_loop(0, nch, chunk, ())

    return gather_diff


def _edge_mlp_body(pre_ref, attr_ref, wea_ref, be_ref, g_ref, bb_ref, o_ref):
    t = pre_ref[...] + jnp.dot(attr_ref[...], wea_ref[...],
                               preferred_element_type=jnp.float32)
    t = jnp.maximum(t + be_ref[...], 0.0)
    m = jnp.mean(t, axis=-1, keepdims=True)
    c = t - m
    v = jnp.mean(c * c, axis=-1, keepdims=True)
    o_ref[...] = c * lax.rsqrt(v + 1e-5) * g_ref[...] + bb_ref[...]


def _make_scatter_sum(N, E, D, CH):
    e_pt = E // NW
    nch = e_pt // CH
    n_pad = ((N + 8 * NS - 1) // (8 * NS)) * (8 * NS)
    n_pt = n_pad // NS
    mesh = plsc.VectorSubcoreMesh(core_axis_name="c", subcore_axis_name="s",
                                  num_cores=NC, num_subcores=NS)

    @functools.partial(
        pl.kernel,
        out_type=[jax.ShapeDtypeStruct((n_pad, D), jnp.float32),
                  jax.ShapeDtypeStruct((n_pad, D), jnp.float32)],
        mesh=mesh,
        scratch_types=[
            pltpu.VMEM((CH,), jnp.int32),
            pltpu.VMEM((CH, D), jnp.float32),
            pltpu.VMEM_SHARED((n_pad, D), jnp.float32),
        ],
    )
    def scatter_sum(feat_hbm, row_hbm, out0_hbm, out1_hbm, idx, buf, acc):
        cid = lax.axis_index("c")
        sid = lax.axis_index("s")
        wid = sid * NC + cid
        base0 = wid * e_pt

        def zero_row(j, _):
            for k in range(D // LANES):
                buf[j, pl.ds(k * LANES, LANES)] = jnp.zeros(
                    (LANES,), jnp.float32)
            return ()

        lax.fori_loop(0, CH, zero_row, ())
        done = 0
        while done < n_pt:
            step = min(CH, n_pt - done)
            pltpu.sync_copy(buf.at[pl.ds(0, step)],
                            acc.at[pl.ds(sid * n_pt + done, step)])
            done += step
        plsc.subcore_barrier()

        def chunk(i, _):
            base = base0 + i * CH
            pltpu.sync_copy(row_hbm.at[pl.ds(base, CH)], idx)
            pltpu.sync_copy(feat_hbm.at[pl.ds(base, CH)], buf)
            pltpu.sync_copy(buf, acc.at[idx], add=True)
            return ()

        lax.fori_loop(0, nch, chunk, ())
        plsc.subcore_barrier()

        @pl.when(cid == 0)
        def _():
            pltpu.sync_copy(acc.at[pl.ds(sid * n_pt, n_pt)],
                            out0_hbm.at[pl.ds(sid * n_pt, n_pt)])

        @pl.when(cid == 1)
        def _():
            pltpu.sync_copy(acc.at[pl.ds(sid * n_pt, n_pt)],
                            out1_hbm.at[pl.ds(sid * n_pt, n_pt)])

    return scatter_sum


def _node_mlp_body(x_ref, p0_ref, p1_ref, wn1_ref, wn2_ref,
                   bn_ref, g_ref, bb_ref, o_ref):
    agg = p0_ref[...] + p1_ref[...]
    h = (jnp.dot(x_ref[...], wn1_ref[...], preferred_element_type=jnp.float32)
         + jnp.dot(agg, wn2_ref[...], preferred_element_type=jnp.float32)
         + bn_ref[...])
    h = jnp.maximum(h, 0.0)
    m = jnp.mean(h, axis=-1, keepdims=True)
    c = h - m
    v = jnp.mean(c * c, axis=-1, keepdims=True)
    o_ref[...] = (c * lax.rsqrt(v + 1e-5) * g_ref[...] + bb_ref[...]
                  + x_ref[...])


def kernel(x, edge_index, edge_attr, W_e, b_e, g_e, be_e,
           W_n, b_n, g_n, be_n):
    N, D = x.shape
    E = edge_index.shape[1]
    DE = edge_attr.shape[1]
    H = W_e.shape[1]
    assert E % NW == 0 and N % NS == 0 and D % LANES == 0

    row = edge_index[0]
    col = edge_index[1]
    W_ex = W_e[:D]
    W_ea = W_e[D:]
    W_n1 = W_n[:D]
    W_n2 = W_n[D:]

    u = pl.pallas_call(
        _proj_body,
        out_shape=jax.ShapeDtypeStruct((N, H), jnp.float32),
    )(x, W_ex)

    CH = 400
    pre = _make_gather_diff(N, E, H, CH)(u, row, col)

    BE = 3200
    grid = (E // BE,)
    edge_feat = pl.pallas_call(
        _edge_mlp_body,
        grid=grid,
        in_specs=[
            pl.BlockSpec((BE, H), lambda i: (i, 0)),
            pl.BlockSpec((BE, DE), lambda i: (i, 0)),
            pl.BlockSpec((DE, H), lambda i: (0, 0)),
            pl.BlockSpec((1, H), lambda i: (0, 0)),
            pl.BlockSpec((1, H), lambda i: (0, 0)),
            pl.BlockSpec((1, H), lambda i: (0, 0)),
        ],
        out_specs=pl.BlockSpec((BE, H), lambda i: (i, 0)),
        out_shape=jax.ShapeDtypeStruct((E, H), jnp.float32),
    )(pre, edge_attr, W_ea,
      b_e.reshape(1, H), g_e.reshape(1, H), be_e.reshape(1, H))

    part0, part1 = _make_scatter_sum(N, E, H, 200)(edge_feat, row)

    BN = N // 5
    x_out = pl.pallas_call(
        _node_mlp_body,
        grid=(5,),
        in_specs=[
            pl.BlockSpec((BN, D), lambda i: (i, 0)),
            pl.BlockSpec((BN, H), lambda i: (i, 0)),
            pl.BlockSpec((BN, H), lambda i: (i, 0)),
            pl.BlockSpec((D, H), lambda i: (0, 0)),
            pl.BlockSpec((H, H), lambda i: (0, 0)),
            pl.BlockSpec((1, H), lambda i: (0, 0)),
            pl.BlockSpec((1, H), lambda i: (0, 0)),
            pl.BlockSpec((1, H), lambda i: (0, 0)),
        ],
        out_specs=pl.BlockSpec((BN, H), lambda i: (i, 0)),
        out_shape=jax.ShapeDtypeStruct((N, H), jnp.float32),
    )(x, part0, part1, W_n1, W_n2,
      b_n.reshape(1, H), g_n.reshape(1, H), be_n.reshape(1, H))

    return (x_out, edge_feat)


# speedup vs baseline: 3.5472x; 1.1984x over previous
import functools

import jax
import jax.numpy as jnp
from jax import lax
from jax.experimental import pallas as pl
from jax.experimental.pallas import tpu as pltpu
from jax.experimental.pallas import tpu_sc as plsc

NC = 2
NS = 16
LANES = 16
NW = NC * NS


def _proj_body(x_ref, w_ref, o_ref):
    o_ref[...] = jnp.dot(x_ref[...], w_ref[...],
                         preferred_element_type=jnp.float32)


def _make_gather_diff(N, E, D, CH):
    e_pt = E // NW
    nch = e_pt // CH
    assert e_pt % CH == 0 and nch >= 3
    mesh = plsc.VectorSubcoreMesh(core_axis_name="c", subcore_axis_name="s",
                                  num_cores=NC, num_subcores=NS)

    @functools.partial(
        pl.kernel,
        out_type=jax.ShapeDtypeStruct((E, D), jnp.float32),
        mesh=mesh,
        scratch_types=[
            pltpu.VMEM((e_pt,), jnp.int32),
            pltpu.VMEM((e_pt,), jnp.int32),
            pltpu.VMEM((CH, D), jnp.float32),
            pltpu.VMEM((CH, D), jnp.float32),
            pltpu.VMEM((CH, D), jnp.float32),
            pltpu.VMEM((CH, D), jnp.float32),
            pltpu.VMEM((CH, D), jnp.float32),
            pltpu.VMEM((CH, D), jnp.float32),
            pltpu.SemaphoreType.DMA,
            pltpu.SemaphoreType.DMA,
            pltpu.SemaphoreType.DMA,
            pltpu.SemaphoreType.DMA,
            pltpu.SemaphoreType.DMA,
            pltpu.SemaphoreType.DMA,
        ],
    )
    def gather_diff(u_hbm, row_hbm, col_hbm, out_hbm, idx_r, idx_c,
                    br0, bc0, ob0, br1, bc1, ob1,
                    sgr0, sgc0, sgr1, sgc1, so0, so1):
        wid = lax.axis_index("s") * NC + lax.axis_index("c")
        base0 = wid * e_pt
        pltpu.sync_copy(row_hbm.at[pl.ds(base0, e_pt)], idx_r)
        pltpu.sync_copy(col_hbm.at[pl.ds(base0, e_pt)], idx_c)
        br = (br0, br1)
        bc = (bc0, bc1)
        ob = (ob0, ob1)
        sgr = (sgr0, sgr1)
        sgc = (sgc0, sgc1)
        so = (so0, so1)

        def issue_gather(chunk, b):
            off = pl.multiple_of(chunk * CH, 8)
            pltpu.async_copy(u_hbm.at[idx_r.at[pl.ds(off, CH)]], br[b], sgr[b])
            pltpu.async_copy(u_hbm.at[idx_c.at[pl.ds(off, CH)]], bc[b], sgc[b])

        def wait_gather(b):
            pltpu.make_async_copy(u_hbm.at[pl.ds(0, CH)], br[b], sgr[b]).wait()
            pltpu.make_async_copy(u_hbm.at[pl.ds(0, CH)], bc[b], sgc[b]).wait()

        def issue_out(chunk, b):
            pltpu.async_copy(ob[b], out_hbm.at[pl.ds(base0 + chunk * CH, CH)],
                             so[b])

        def wait_out(b):
            pltpu.make_async_copy(u_hbm.at[pl.ds(0, CH)], ob[b], so[b]).wait()

        def sub(b):
            def _row(j, _):
                for k in range(D // LANES):
                    sl = pl.ds(k * LANES, LANES)
                    ob[b][j, sl] = br[b][j, sl] - bc[b][j, sl]
                return ()

            lax.fori_loop(0, CH, _row, (), unroll=2)

        issue_gather(0, 0)

        def body(i, _):
            g = i * 2

            @pl.when(g + 1 < nch)
            def _():
                issue_gather(g + 1, 1)

            @pl.when(g >= 2)
            def _():
                wait_out(0)

            wait_gather(0)
            sub(0)
            issue_out(g, 0)

            @pl.when(g + 2 < nch)
            def _():
                issue_gather(g + 2, 0)

            @pl.when(g + 1 < nch)
            def _():
                @pl.when(g >= 2)
                def _():
                    wait_out(1)

                wait_gather(1)
                sub(1)
                issue_out(g + 1, 1)

            return ()

        lax.fori_loop(0, (nch + 1) // 2, body, ())
        wait_out(0)
        wait_out(1)

    return gather_diff


def _edge_mlp_body(pre_ref, attr_ref, wea_ref, be_ref, g_ref, bb_ref, o_ref):
    t = pre_ref[...] + jnp.dot(attr_ref[...], wea_ref[...],
                               preferred_element_type=jnp.float32)
    t = jnp.maximum(t + be_ref[...], 0.0)
    m = jnp.mean(t, axis=-1, keepdims=True)
    c = t - m
    v = jnp.mean(c * c, axis=-1, keepdims=True)
    o_ref[...] = c * lax.rsqrt(v + 1e-5) * g_ref[...] + bb_ref[...]


def _make_scatter_sum(N, E, D, CH):
    e_pt = E // NW
    nch = e_pt // CH
    n_pad = ((N + 8 * NS - 1) // (8 * NS)) * (8 * NS)
    n_pt = n_pad // NS
    mesh = plsc.VectorSubcoreMesh(core_axis_name="c", subcore_axis_name="s",
                                  num_cores=NC, num_subcores=NS)

    @functools.partial(
        pl.kernel,
        out_type=[jax.ShapeDtypeStruct((n_pad, D), jnp.float32),
                  jax.ShapeDtypeStruct((n_pad, D), jnp.float32)],
        mesh=mesh,
        scratch_types=[
            pltpu.VMEM((CH,), jnp.int32),
            pltpu.VMEM((CH, D), jnp.float32),
            pltpu.VMEM_SHARED((n_pad, D), jnp.float32),
        ],
    )
    def scatter_sum(feat_hbm, row_hbm, out0_hbm, out1_hbm, idx, buf, acc):
        cid = lax.axis_index("c")
        sid = lax.axis_index("s")
        wid = sid * NC + cid
        base0 = wid * e_pt

        def zero_row(j, _):
            for k in range(D // LANES):
                buf[j, pl.ds(k * LANES, LANES)] = jnp.zeros(
                    (LANES,), jnp.float32)
            return ()

        lax.fori_loop(0, CH, zero_row, ())
        done = 0
        while done < n_pt:
            step = min(CH, n_pt - done)
            pltpu.sync_copy(buf.at[pl.ds(0, step)],
                            acc.at[pl.ds(sid * n_pt + done, step)])
            done += step
        plsc.subcore_barrier()

        def chunk(i, _):
            base = base0 + i * CH
            pltpu.sync_copy(row_hbm.at[pl.ds(base, CH)], idx)
            pltpu.sync_copy(feat_hbm.at[pl.ds(base, CH)], buf)
            pltpu.sync_copy(buf, acc.at[idx], add=True)
            return ()

        lax.fori_loop(0, nch, chunk, ())
        plsc.subcore_barrier()

        @pl.when(cid == 0)
        def _():
            pltpu.sync_copy(acc.at[pl.ds(sid * n_pt, n_pt)],
                            out0_hbm.at[pl.ds(sid * n_pt, n_pt)])

        @pl.when(cid == 1)
        def _():
            pltpu.sync_copy(acc.at[pl.ds(sid * n_pt, n_pt)],
                            out1_hbm.at[pl.ds(sid * n_pt, n_pt)])

    return scatter_sum


def _node_mlp_body(x_ref, p0_ref, p1_ref, wn1_ref, wn2_ref,
                   bn_ref, g_ref, bb_ref, o_ref):
    agg = p0_ref[...] + p1_ref[...]
    h = (jnp.dot(x_ref[...], wn1_ref[...], preferred_element_type=jnp.float32)
         + jnp.dot(agg, wn2_ref[...], preferred_element_type=jnp.float32)
         + bn_ref[...])
    h = jnp.maximum(h, 0.0)
    m = jnp.mean(h, axis=-1, keepdims=True)
    c = h - m
    v = jnp.mean(c * c, axis=-1, keepdims=True)
    o_ref[...] = (c * lax.rsqrt(v + 1e-5) * g_ref[...] + bb_ref[...]
                  + x_ref[...])


def kernel(x, edge_index, edge_attr, W_e, b_e, g_e, be_e,
           W_n, b_n, g_n, be_n):
    N, D = x.shape
    E = edge_index.shape[1]
    DE = edge_attr.shape[1]
    H = W_e.shape[1]
    assert E % NW == 0 and N % NS == 0 and D % LANES == 0

    row = edge_index[0]
    col = edge_index[1]
    W_ex = W_e[:D]
    W_ea = W_e[D:]
    W_n1 = W_n[:D]
    W_n2 = W_n[D:]

    u = pl.pallas_call(
        _proj_body,
        out_shape=jax.ShapeDtypeStruct((N, H), jnp.float32),
    )(x, W_ex)

    pre = _make_gather_diff(N, E, H, 80)(u, row, col)

    BE = 3200
    grid = (E // BE,)
    edge_feat = pl.pallas_call(
        _edge_mlp_body,
        grid=grid,
        in_specs=[
            pl.BlockSpec((BE, H), lambda i: (i, 0)),
            pl.BlockSpec((BE, DE), lambda i: (i, 0)),
            pl.BlockSpec((DE, H), lambda i: (0, 0)),
            pl.BlockSpec((1, H), lambda i: (0, 0)),
            pl.BlockSpec((1, H), lambda i: (0, 0)),
            pl.BlockSpec((1, H), lambda i: (0, 0)),
        ],
        out_specs=pl.BlockSpec((BE, H), lambda i: (i, 0)),
        out_shape=jax.ShapeDtypeStruct((E, H), jnp.float32),
    )(pre, edge_attr, W_ea,
      b_e.reshape(1, H), g_e.reshape(1, H), be_e.reshape(1, H))

    part0, part1 = _make_scatter_sum(N, E, H, 200)(edge_feat, row)

    BN = N // 5
    x_out = pl.pallas_call(
        _node_mlp_body,
        grid=(5,),
        in_specs=[
            pl.BlockSpec((BN, D), lambda i: (i, 0)),
            pl.BlockSpec((BN, H), lambda i: (i, 0)),
            pl.BlockSpec((BN, H), lambda i: (i, 0)),
            pl.BlockSpec((D, H), lambda i: (0, 0)),
            pl.BlockSpec((H, H), lambda i: (0, 0)),
            pl.BlockSpec((1, H), lambda i: (0, 0)),
            pl.BlockSpec((1, H), lambda i: (0, 0)),
            pl.BlockSpec((1, H), lambda i: (0, 0)),
        ],
        out_specs=pl.BlockSpec((BN, H), lambda i: (i, 0)),
        out_shape=jax.ShapeDtypeStruct((N, H), jnp.float32),
    )(x, part0, part1, W_n1, W_n2,
      b_n.reshape(1, H), g_n.reshape(1, H), be_n.reshape(1, H))

    return (x_out, edge_feat)
